# Initial kernel scaffold; baseline (speedup 1.0000x reference)
import functools

import jax
import jax.numpy as jnp
from jax import lax
from jax.experimental import pallas as pl
from jax.experimental.pallas import tpu as pltpu
from jax.experimental.pallas import tpu_sc as plsc

N = 100000
E = 6400000
D = 2
C_CONST = 1.0
R_C = 1.0
P_POW = 2
GAMMA = 0.1

NC = 2
NS = 16
LANES = 16
NW = NC * NS
NPAD = 102400
SL = NPAD // NS
EPW = E // NW
CH = 2000
NCHUNK = EPW // CH
RSQRT_MAGIC = jnp.int32(0x5F3759DF)


def _fast_rsqrt(s):
    bits = plsc.bitcast(s, jnp.int32)
    y = plsc.bitcast(RSQRT_MAGIC - lax.shift_right_logical(bits, 1),
                     jnp.float32)
    half_s = 0.5 * s
    for _ in range(3):
        y = y * (1.5 - half_s * y * y)
    return y


def _sc_body(x0_hbm, x1_hbm, src_hbm, dst_hbm, p0_hbm, p1_hbm,
             srcv, dstv, xs0, xs1, xd0, xd1, m0v, m1v, tmpv,
             x0sh, x1sh, a0sh, a1sh):
    cid = lax.axis_index("c")
    sid = lax.axis_index("s")
    wid = sid * NC + cid
    nbase = sid * SL

    pltpu.sync_copy(x0_hbm.at[pl.ds(nbase, SL)], tmpv)
    pltpu.sync_copy(tmpv, x0sh.at[pl.ds(nbase, SL)])
    pltpu.sync_copy(x1_hbm.at[pl.ds(nbase, SL)], tmpv)
    pltpu.sync_copy(tmpv, x1sh.at[pl.ds(nbase, SL)])

    def zero_body(i, _):
        tmpv[pl.ds(i * LANES, LANES)] = jnp.zeros((LANES,), jnp.float32)
        return 0
    lax.fori_loop(0, SL // LANES, zero_body, 0)
    pltpu.sync_copy(tmpv, a0sh.at[pl.ds(nbase, SL)])
    pltpu.sync_copy(tmpv, a1sh.at[pl.ds(nbase, SL)])

    plsc.subcore_barrier()

    ebase = wid * EPW

    def chunk_body(ci, _):
        off = ebase + ci * CH
        pltpu.sync_copy(src_hbm.at[pl.ds(off, CH)], srcv)
        pltpu.sync_copy(dst_hbm.at[pl.ds(off, CH)], dstv)
        pltpu.sync_copy(x0sh.at[srcv], xs0)
        pltpu.sync_copy(x1sh.at[srcv], xs1)
        pltpu.sync_copy(x0sh.at[dstv], xd0)
        pltpu.sync_copy(x1sh.at[dstv], xd1)

        def vec_body(i, _):
            sl = pl.ds(i * LANES, LANES)
            d0 = xd0[sl] - xs0[sl]
            d1 = xd1[sl] - xs1[sl]
            s = d0 * d0 + d1 * d1
            y = _fast_rsqrt(s)
            g = (C_CONST * P_POW * R_C) * y - (C_CONST * P_POW)
            m0v[sl] = g * d0
            m1v[sl] = g * d1
            return 0
        lax.fori_loop(0, CH // LANES, vec_body, 0)

        pltpu.sync_copy(m0v, a0sh.at[dstv], add=True)
        pltpu.sync_copy(m1v, a1sh.at[dstv], add=True)
        return 0

    lax.fori_loop(0, NCHUNK, chunk_body, 0)

    plsc.subcore_barrier()

    obase = cid * NPAD + nbase
    pltpu.sync_copy(a0sh.at[pl.ds(nbase, SL)], tmpv)
    pltpu.sync_copy(tmpv, p0_hbm.at[pl.ds(obase, SL)])
    pltpu.sync_copy(a1sh.at[pl.ds(nbase, SL)], tmpv)
    pltpu.sync_copy(tmpv, p1_hbm.at[pl.ds(obase, SL)])


@jax.jit
def _sc_partials(x0p, x1p, src, dst):
    mesh = plsc.VectorSubcoreMesh(core_axis_name="c", subcore_axis_name="s")
    f = pl.kernel(
        _sc_body,
        out_type=[
            jax.ShapeDtypeStruct((NC * NPAD,), jnp.float32),
            jax.ShapeDtypeStruct((NC * NPAD,), jnp.float32),
        ],
        mesh=mesh,
        scratch_types=[
            pltpu.VMEM((CH,), jnp.int32),
            pltpu.VMEM((CH,), jnp.int32),
            pltpu.VMEM((CH,), jnp.float32),
            pltpu.VMEM((CH,), jnp.float32),
            pltpu.VMEM((CH,), jnp.float32),
            pltpu.VMEM((CH,), jnp.float32),
            pltpu.VMEM((CH,), jnp.float32),
            pltpu.VMEM((CH,), jnp.float32),
            pltpu.VMEM((SL,), jnp.float32),
            pltpu.VMEM_SHARED((NPAD,), jnp.float32),
            pltpu.VMEM_SHARED((NPAD,), jnp.float32),
            pltpu.VMEM_SHARED((NPAD,), jnp.float32),
            pltpu.VMEM_SHARED((NPAD,), jnp.float32),
        ],
    )
    return f(x0p, x1p, src, dst)


def _combine_body(p00, p01, p10, p11, v0, v1, a0, a1):
    a0[...] = p00[...] + p01[...] - GAMMA * v0[...]
    a1[...] = p10[...] + p11[...] - GAMMA * v1[...]


@jax.jit
def _combine(p0, p1, v0p, v1p):
    rows = NPAD // 128
    args = (
        p0[:NPAD].reshape(rows, 128),
        p0[NPAD:].reshape(rows, 128),
        p1[:NPAD].reshape(rows, 128),
        p1[NPAD:].reshape(rows, 128),
        v0p.reshape(rows, 128),
        v1p.reshape(rows, 128),
    )
    out = pl.pallas_call(
        _combine_body,
        out_shape=[
            jax.ShapeDtypeStruct((rows, 128), jnp.float32),
            jax.ShapeDtypeStruct((rows, 128), jnp.float32),
        ],
    )(*args)
    return out[0].reshape(-1), out[1].reshape(-1)


def kernel(x, v, edge_index):
    src = edge_index[0]
    dst = edge_index[1]
    x0p = jnp.zeros((NPAD,), jnp.float32).at[:N].set(x[:, 0])
    x1p = jnp.zeros((NPAD,), jnp.float32).at[:N].set(x[:, 1])
    v0p = jnp.zeros((NPAD,), jnp.float32).at[:N].set(v[:, 0])
    v1p = jnp.zeros((NPAD,), jnp.float32).at[:N].set(v[:, 1])
    p0, p1 = _sc_partials(x0p, x1p, src, dst)
    a0, a1 = _combine(p0, p1, v0p, v1p)
    return jnp.stack([a0[:N], a1[:N]], axis=-1)


# speedup vs baseline: 86.5291x; 86.5291x over previous
import functools

import jax
import jax.numpy as jnp
from jax import lax
from jax.experimental import pallas as pl
from jax.experimental.pallas import tpu as pltpu
from jax.experimental.pallas import tpu_sc as plsc

N = 100000
E = 6400000
D = 2
C_CONST = 1.0
R_C = 1.0
P_POW = 2
GAMMA = 0.1

NC = 2
NS = 16
LANES = 16
NW = NC * NS
NPAD = 102400
SL = NPAD // NS
EPW = E // NW
CH = 2000
NCHUNK = EPW // CH
RSQRT_MAGIC = 0x5F3759DF


def _fast_rsqrt(s):
    bits = lax.bitcast_convert_type(s, jnp.int32)
    y = lax.bitcast_convert_type(
        RSQRT_MAGIC - lax.shift_right_logical(bits, 1), jnp.float32)
    half_s = 0.5 * s
    for _ in range(3):
        y = y * (1.5 - half_s * y * y)
    return y


def _sc_body(x0_hbm, x1_hbm, src_hbm, dst_hbm, p0_hbm, p1_hbm,
             srcv, dstv, xs0, xs1, xd0, xd1, m0v, m1v, tmpv,
             x0sh, x1sh, a0sh, a1sh):
    cid = lax.axis_index("c")
    sid = lax.axis_index("s")
    wid = sid * NC + cid
    nbase = sid * SL

    pltpu.sync_copy(x0_hbm.at[pl.ds(nbase, SL)], tmpv)
    pltpu.sync_copy(tmpv, x0sh.at[pl.ds(nbase, SL)])
    pltpu.sync_copy(x1_hbm.at[pl.ds(nbase, SL)], tmpv)
    pltpu.sync_copy(tmpv, x1sh.at[pl.ds(nbase, SL)])

    def zero_body(i, _):
        tmpv[pl.ds(i * LANES, LANES)] = jnp.zeros((LANES,), jnp.float32)
        return 0
    lax.fori_loop(0, SL // LANES, zero_body, 0)
    pltpu.sync_copy(tmpv, a0sh.at[pl.ds(nbase, SL)])
    pltpu.sync_copy(tmpv, a1sh.at[pl.ds(nbase, SL)])

    plsc.subcore_barrier()

    ebase = wid * EPW

    def chunk_body(ci, _):
        off = ebase + ci * CH
        pltpu.sync_copy(src_hbm.at[pl.ds(off, CH)], srcv)
        pltpu.sync_copy(dst_hbm.at[pl.ds(off, CH)], dstv)
        pltpu.sync_copy(x0sh.at[srcv], xs0)
        pltpu.sync_copy(x1sh.at[srcv], xs1)
        pltpu.sync_copy(x0sh.at[dstv], xd0)
        pltpu.sync_copy(x1sh.at[dstv], xd1)

        def vec_body(i, _):
            sl = pl.ds(i * LANES, LANES)
            d0 = xd0[sl] - xs0[sl]
            d1 = xd1[sl] - xs1[sl]
            s = d0 * d0 + d1 * d1
            y = _fast_rsqrt(s)
            g = (C_CONST * P_POW * R_C) * y - (C_CONST * P_POW)
            m0v[sl] = g * d0
            m1v[sl] = g * d1
            return 0
        lax.fori_loop(0, CH // LANES, vec_body, 0)

        pltpu.sync_copy(m0v, a0sh.at[dstv], add=True)
        pltpu.sync_copy(m1v, a1sh.at[dstv], add=True)
        return 0

    lax.fori_loop(0, NCHUNK, chunk_body, 0)

    plsc.subcore_barrier()

    obase = cid * NPAD + nbase
    pltpu.sync_copy(a0sh.at[pl.ds(nbase, SL)], tmpv)
    pltpu.sync_copy(tmpv, p0_hbm.at[pl.ds(obase, SL)])
    pltpu.sync_copy(a1sh.at[pl.ds(nbase, SL)], tmpv)
    pltpu.sync_copy(tmpv, p1_hbm.at[pl.ds(obase, SL)])


@jax.jit
def _sc_partials(x0p, x1p, src, dst):
    mesh = plsc.VectorSubcoreMesh(core_axis_name="c", subcore_axis_name="s")
    f = pl.kernel(
        _sc_body,
        out_type=[
            jax.ShapeDtypeStruct((NC * NPAD,), jnp.float32),
            jax.ShapeDtypeStruct((NC * NPAD,), jnp.float32),
        ],
        mesh=mesh,
        scratch_types=[
            pltpu.VMEM((CH,), jnp.int32),
            pltpu.VMEM((CH,), jnp.int32),
            pltpu.VMEM((CH,), jnp.float32),
            pltpu.VMEM((CH,), jnp.float32),
            pltpu.VMEM((CH,), jnp.float32),
            pltpu.VMEM((CH,), jnp.float32),
            pltpu.VMEM((CH,), jnp.float32),
            pltpu.VMEM((CH,), jnp.float32),
            pltpu.VMEM((SL,), jnp.float32),
            pltpu.VMEM_SHARED((NPAD,), jnp.float32),
            pltpu.VMEM_SHARED((NPAD,), jnp.float32),
            pltpu.VMEM_SHARED((NPAD,), jnp.float32),
            pltpu.VMEM_SHARED((NPAD,), jnp.float32),
        ],
    )
    return f(x0p, x1p, src, dst)


def _combine_body(p00, p01, p10, p11, v0, v1, a0, a1):
    a0[...] = p00[...] + p01[...] - GAMMA * v0[...]
    a1[...] = p10[...] + p11[...] - GAMMA * v1[...]


@jax.jit
def _combine(p0, p1, v0p, v1p):
    rows = NPAD // 128
    args = (
        p0[:NPAD].reshape(rows, 128),
        p0[NPAD:].reshape(rows, 128),
        p1[:NPAD].reshape(rows, 128),
        p1[NPAD:].reshape(rows, 128),
        v0p.reshape(rows, 128),
        v1p.reshape(rows, 128),
    )
    out = pl.pallas_call(
        _combine_body,
        out_shape=[
            jax.ShapeDtypeStruct((rows, 128), jnp.float32),
            jax.ShapeDtypeStruct((rows, 128), jnp.float32),
        ],
    )(*args)
    return out[0].reshape(-1), out[1].reshape(-1)


def kernel(x, v, edge_index):
    src = edge_index[0]
    dst = edge_index[1]
    x0p = jnp.zeros((NPAD,), jnp.float32).at[:N].set(x[:, 0])
    x1p = jnp.zeros((NPAD,), jnp.float32).at[:N].set(x[:, 1])
    v0p = jnp.zeros((NPAD,), jnp.float32).at[:N].set(v[:, 0])
    v1p = jnp.zeros((NPAD,), jnp.float32).at[:N].set(v[:, 1])
    p0, p1 = _sc_partials(x0p, x1p, src, dst)
    a0, a1 = _combine(p0, p1, v0p, v1p)
    return jnp.stack([a0[:N], a1[:N]], axis=-1)


# speedup vs baseline: 135.3825x; 1.5646x over previous
import jax
import jax.numpy as jnp
from jax import lax
from jax.experimental import pallas as pl
from jax.experimental.pallas import tpu as pltpu
from jax.experimental.pallas import tpu_sc as plsc

N = 100000
E = 6400000
C_CONST = 1.0
R_C = 1.0
P_POW = 2
GAMMA = 0.1

NC = 2
NS = 16
LANES = 16
NW = NC * NS
NPAD = 102400
SL = NPAD // NS
EPW = E // NW
CH = 2000
NCHUNK = EPW // CH
NOUT = NCHUNK // 4
RSQRT_MAGIC = 0x5F3759DF


def _fast_rsqrt(s):
    bits = lax.bitcast_convert_type(s, jnp.int32)
    y = lax.bitcast_convert_type(
        RSQRT_MAGIC - lax.shift_right_logical(bits, 1), jnp.float32)
    half_s = 0.5 * s
    for _ in range(2):
        y = y * (1.5 - half_s * y * y)
    return y


def _sc_body(x0_hbm, x1_hbm, src_hbm, dst_hbm, p0_hbm, p1_hbm,
             idxs0, idxs1, idxs2, idxs3, idxd0, idxd1, idxd2, idxd3,
             xs0a, xs0b, xs1a, xs1b, xd0a, xd0b, xd1a, xd1b,
             m0a, m0b, m1a, m1b, tmpv,
             x0sh, x1sh, a0sh, a1sh, sem_i, sem_g, sem_s):
    idxs = [idxs0, idxs1, idxs2, idxs3]
    idxd = [idxd0, idxd1, idxd2, idxd3]
    xs0 = [xs0a, xs0b]
    xs1 = [xs1a, xs1b]
    xd0 = [xd0a, xd0b]
    xd1 = [xd1a, xd1b]
    m0 = [m0a, m0b]
    m1 = [m1a, m1b]

    cid = lax.axis_index("c")
    sid = lax.axis_index("s")
    wid = sid * NC + cid
    nbase = sid * SL

    pltpu.sync_copy(x0_hbm.at[pl.ds(nbase, SL)], tmpv)
    pltpu.sync_copy(tmpv, x0sh.at[pl.ds(nbase, SL)])
    pltpu.sync_copy(x1_hbm.at[pl.ds(nbase, SL)], tmpv)
    pltpu.sync_copy(tmpv, x1sh.at[pl.ds(nbase, SL)])

    def zero_body(i, _):
        tmpv[pl.ds(i * LANES, LANES)] = jnp.zeros((LANES,), jnp.float32)
        return 0
    lax.fori_loop(0, SL // LANES, zero_body, 0, unroll=8)
    pltpu.sync_copy(tmpv, a0sh.at[pl.ds(nbase, SL)])
    pltpu.sync_copy(tmpv, a1sh.at[pl.ds(nbase, SL)])

    plsc.subcore_barrier()

    ebase = wid * EPW

    def chunk_off(ci):
        return jnp.minimum(ebase + ci * CH, E - CH)

    def idx_start(ci, bi):
        off = chunk_off(ci)
        pltpu.async_copy(src_hbm.at[pl.ds(off, CH)], idxs[bi], sem_i.at[bi])
        pltpu.async_copy(dst_hbm.at[pl.ds(off, CH)], idxd[bi], sem_i.at[bi])

    def idx_wait(bi):
        pltpu.make_async_copy(
            src_hbm.at[pl.ds(0, CH)], idxs[bi], sem_i.at[bi]).wait()
        pltpu.make_async_copy(
            dst_hbm.at[pl.ds(0, CH)], idxd[bi], sem_i.at[bi]).wait()

    def gather_start(bi, bg):
        pltpu.async_copy(x0sh.at[idxs[bi]], xs0[bg], sem_g.at[bg])
        pltpu.async_copy(x1sh.at[idxs[bi]], xs1[bg], sem_g.at[bg])
        pltpu.async_copy(x0sh.at[idxd[bi]], xd0[bg], sem_g.at[bg])
        pltpu.async_copy(x1sh.at[idxd[bi]], xd1[bg], sem_g.at[bg])

    def gather_wait(bg):
        for bufs in (xs0, xs1, xd0, xd1):
            pltpu.make_async_copy(
                x0_hbm.at[pl.ds(0, CH)], bufs[bg], sem_g.at[bg]).wait()

    def compute(bg):
        def vec_body(i, _):
            sl = pl.ds(i * LANES, LANES)
            d0 = xd0[bg][sl] - xs0[bg][sl]
            d1 = xd1[bg][sl] - xs1[bg][sl]
            s = d0 * d0 + d1 * d1
            y = _fast_rsqrt(s)
            g = (C_CONST * P_POW * R_C) * y - (C_CONST * P_POW)
            m0[bg][sl] = g * d0
            m1[bg][sl] = g * d1
            return 0
        lax.fori_loop(0, CH // LANES, vec_body, 0, unroll=5)

    def scatter_start(bi, bg):
        pltpu.async_copy(m0[bg], a0sh.at[idxd[bi]], sem_s.at[bg], add=True)
        pltpu.async_copy(m1[bg], a1sh.at[idxd[bi]], sem_s.at[bg], add=True)

    def scatter_wait(bg):
        for bufs in (m0, m1):
            pltpu.make_async_copy(
                x0_hbm.at[pl.ds(0, CH)], bufs[bg], sem_s.at[bg]).wait()

    def step(ci, u, skip_scatter_wait):
        bg = u % 2
        if not skip_scatter_wait:
            scatter_wait(bg)
        idx_start(ci + 2, (u + 2) % 4)
        gather_wait(bg)
        idx_wait((u + 1) % 4)
        gather_start((u + 1) % 4, (u + 1) % 2)
        compute(bg)
        scatter_start(u, bg)

    idx_start(0, 0)
    idx_start(1, 1)
    idx_wait(0)
    gather_start(0, 0)

    for u in range(4):
        step(jnp.int32(u), u, skip_scatter_wait=(u < 2))

    def outer_body(j, _):
        ci0 = j * 4
        for u in range(4):
            step(ci0 + u, u, skip_scatter_wait=False)
        return 0
    lax.fori_loop(1, NOUT, outer_body, 0)

    scatter_wait(0)
    scatter_wait(1)
    gather_wait(NCHUNK % 2)
    idx_wait((NCHUNK + 1) % 4)

    plsc.subcore_barrier()

    obase = cid * NPAD + nbase
    pltpu.sync_copy(a0sh.at[pl.ds(nbase, SL)], tmpv)
    pltpu.sync_copy(tmpv, p0_hbm.at[pl.ds(obase, SL)])
    pltpu.sync_copy(a1sh.at[pl.ds(nbase, SL)], tmpv)
    pltpu.sync_copy(tmpv, p1_hbm.at[pl.ds(obase, SL)])


@jax.jit
def _sc_partials(x0p, x1p, src, dst):
    mesh = plsc.VectorSubcoreMesh(core_axis_name="c", subcore_axis_name="s")
    iscr = [pltpu.VMEM((CH,), jnp.int32) for _ in range(8)]
    fscr = [pltpu.VMEM((CH,), jnp.float32) for _ in range(12)]
    f = pl.kernel(
        _sc_body,
        out_type=[
            jax.ShapeDtypeStruct((NC * NPAD,), jnp.float32),
            jax.ShapeDtypeStruct((NC * NPAD,), jnp.float32),
        ],
        mesh=mesh,
        scratch_types=iscr + fscr + [
            pltpu.VMEM((SL,), jnp.float32),
            pltpu.VMEM_SHARED((NPAD,), jnp.float32),
            pltpu.VMEM_SHARED((NPAD,), jnp.float32),
            pltpu.VMEM_SHARED((NPAD,), jnp.float32),
            pltpu.VMEM_SHARED((NPAD,), jnp.float32),
            pltpu.SemaphoreType.DMA((4,)),
            pltpu.SemaphoreType.DMA((2,)),
            pltpu.SemaphoreType.DMA((2,)),
        ],
    )
    return f(x0p, x1p, src, dst)


def _combine_body(p00, p01, p10, p11, v0, v1, a0, a1):
    a0[...] = p00[...] + p01[...] - GAMMA * v0[...]
    a1[...] = p10[...] + p11[...] - GAMMA * v1[...]


@jax.jit
def _combine(p0, p1, v0p, v1p):
    rows = NPAD // 128
    args = (
        p0[:NPAD].reshape(rows, 128),
        p0[NPAD:].reshape(rows, 128),
        p1[:NPAD].reshape(rows, 128),
        p1[NPAD:].reshape(rows, 128),
        v0p.reshape(rows, 128),
        v1p.reshape(rows, 128),
    )
    out = pl.pallas_call(
        _combine_body,
        out_shape=[
            jax.ShapeDtypeStruct((rows, 128), jnp.float32),
            jax.ShapeDtypeStruct((rows, 128), jnp.float32),
        ],
    )(*args)
    return out[0].reshape(-1), out[1].reshape(-1)


def kernel(x, v, edge_index):
    src = edge_index[0]
    dst = edge_index[1]
    x0p = jnp.zeros((NPAD,), jnp.float32).at[:N].set(x[:, 0])
    x1p = jnp.zeros((NPAD,), jnp.float32).at[:N].set(x[:, 1])
    v0p = jnp.zeros((NPAD,), jnp.float32).at[:N].set(v[:, 0])
    v1p = jnp.zeros((NPAD,), jnp.float32).at[:N].set(v[:, 1])
    p0, p1 = _sc_partials(x0p, x1p, src, dst)
    a0, a1 = _combine(p0, p1, v0p, v1p)
    return jnp.stack([a0[:N], a1[:N]], axis=-1)
